# baseline (speedup 1.0000x reference)
import functools

import jax
import jax.numpy as jnp
from jax import lax
from jax.experimental import pallas as pl
from jax.experimental.pallas import tpu as pltpu
import jax.experimental.pallas.tpu_sc as plsc

E = 320000
N = 10000
D = 128
DE = 16
HIDDEN = 128

NC = 2
NS = 16
CHUNK = 128
CHUNKS_PER_W = 20
EDGES_PER_W = CHUNK * CHUNKS_PER_W
SLICE_E = NC * NS * EDGES_PER_W
NSLICES = 4
E_PAD = NSLICES * SLICE_E
ROWS_PER_TILE = 632
N_PAD = NS * ROWS_PER_TILE
MSG_CLAMP = E - CHUNK
LANE = 16

BLK = 16384


def _mlp_body(xi_ref, xj_ref, e_ref, w1a_ref, w1b_ref, w1c_ref,
              b1_ref, w2_ref, b2_ref, out_ref):
    h = jnp.dot(xj_ref[...], w1a_ref[...], preferred_element_type=jnp.float32)
    h = h + jnp.dot(e_ref[...], w1b_ref[...], preferred_element_type=jnp.float32)
    h = h + jnp.dot(xi_ref[...], w1c_ref[...], preferred_element_type=jnp.float32)
    h = h + b1_ref[...]
    h = jnp.where(h >= 0, h, 0.01 * h)
    rows = [
        lax.dot_general(
            w2_ref[...], h[r * CHUNK:(r + 1) * CHUNK, :],
            (((1,), (1,)), ((), ())), preferred_element_type=jnp.float32)
        for r in range(BLK // CHUNK)
    ]
    w = jnp.concatenate(rows, axis=0) + b2_ref[0, 0]
    out_ref[...] = jnp.tanh(w)


BLKS_PER_SLICE = SLICE_E // BLK


def _gate_mlp(sl, x_i, x_j, e_ij, w1a, w1b, w1c, b1r, w2r, b2r):
    off = sl * BLKS_PER_SLICE
    edge = lambda i: (i + off, 0)
    const = lambda i: (0, 0)
    return pl.pallas_call(
        _mlp_body,
        grid=(BLKS_PER_SLICE,),
        in_specs=[
            pl.BlockSpec((BLK, D), edge),
            pl.BlockSpec((BLK, D), edge),
            pl.BlockSpec((BLK, DE), edge),
            pl.BlockSpec((D, HIDDEN), const),
            pl.BlockSpec((DE, HIDDEN), const),
            pl.BlockSpec((D, HIDDEN), const),
            pl.BlockSpec((1, HIDDEN), const),
            pl.BlockSpec((1, HIDDEN), const),
            pl.BlockSpec((1, 1), const),
        ],
        out_specs=pl.BlockSpec((BLK // CHUNK, CHUNK), lambda i: (i, 0)),
        out_shape=jax.ShapeDtypeStruct((SLICE_E // CHUNK, CHUNK), jnp.float32),
    )(x_i, x_j, e_ij, w1a, w1b, w1c, b1r, w2r, b2r)


def _sc_scatter_body(slice_base, msg_hbm, w_hbm, idx_hbm, zrows_hbm, out_hbm,
                     idxbuf, m0, m1, w0, w1, sm0, sm1, sw0, sw1, accum):
    c = lax.axis_index("c")
    s = lax.axis_index("s")
    wkr = c * NS + s
    pltpu.sync_copy(idx_hbm.at[wkr], idxbuf)
    pltpu.sync_copy(zrows_hbm, accum.at[pl.ds(s * ROWS_PER_TILE, ROWS_PER_TILE)])
    plsc.subcore_barrier()

    row0 = slice_base + wkr * EDGES_PER_W

    def mrow(j):
        return jnp.minimum(row0 + j * CHUNK, MSG_CLAMP)

    def start(j, mbuf, wbuf, msem, wsem):
        pltpu.async_copy(msg_hbm.at[pl.ds(mrow(j), CHUNK)], mbuf, msem)
        pltpu.async_copy(w_hbm.at[wkr * CHUNKS_PER_W + j], wbuf, wsem)

    def wait(j, mbuf, wbuf, msem, wsem):
        pltpu.make_async_copy(
            msg_hbm.at[pl.ds(mrow(j), CHUNK)], mbuf, msem).wait()
        pltpu.make_async_copy(
            w_hbm.at[wkr * CHUNKS_PER_W + j], wbuf, wsem).wait()

    def apply_gate(mbuf, wbuf):
        def group(g, carry):
            r0 = g * LANE
            wv = wbuf[pl.ds(r0, LANE)]
            for i in range(LANE):
                splat = lax.gather(
                    wv, jnp.full((LANE, 1), i, jnp.int32),
                    lax.GatherDimensionNumbers(
                        offset_dims=(), collapsed_slice_dims=(0,),
                        start_index_map=(0,)),
                    (1,), mode=lax.GatherScatterMode.PROMISE_IN_BOUNDS)
                for k in range(D // LANE):
                    sl = pl.ds(k * LANE, LANE)
                    mbuf[r0 + i, sl] = mbuf[r0 + i, sl] * splat
            return carry

        lax.fori_loop(0, CHUNK // LANE, group, 0)

    def scatter(j, mbuf):
        pltpu.sync_copy(mbuf, accum.at[idxbuf.at[j]], add=True)

    start(0, m0, w0, sm0, sw0)

    def body(q, carry):
        j = 2 * q
        start(j + 1, m1, w1, sm1, sw1)
        wait(j, m0, w0, sm0, sw0)
        apply_gate(m0, w0)
        scatter(j, m0)
        start(j + 2, m0, w0, sm0, sw0)
        wait(j + 1, m1, w1, sm1, sw1)
        apply_gate(m1, w1)
        scatter(j + 1, m1)
        return carry

    lax.fori_loop(0, CHUNKS_PER_W // 2 - 1, body, 0)
    start(CHUNKS_PER_W - 1, m1, w1, sm1, sw1)
    wait(CHUNKS_PER_W - 2, m0, w0, sm0, sw0)
    apply_gate(m0, w0)
    scatter(CHUNKS_PER_W - 2, m0)
    wait(CHUNKS_PER_W - 1, m1, w1, sm1, sw1)
    apply_gate(m1, w1)
    scatter(CHUNKS_PER_W - 1, m1)

    plsc.subcore_barrier()
    pltpu.sync_copy(accum.at[pl.ds(s * ROWS_PER_TILE, ROWS_PER_TILE)],
                    out_hbm.at[c, pl.ds(s * ROWS_PER_TILE, ROWS_PER_TILE)])


def _sc_scatter_call(sl):
  return functools.partial(
    pl.kernel,
    out_type=jax.ShapeDtypeStruct((NC, N_PAD, D), jnp.float32),
    mesh=plsc.VectorSubcoreMesh(core_axis_name="c", subcore_axis_name="s"),
    scratch_types=[
        pltpu.VMEM((CHUNKS_PER_W, CHUNK), jnp.int32),
        pltpu.VMEM((CHUNK, D), jnp.float32),
        pltpu.VMEM((CHUNK, D), jnp.float32),
        pltpu.VMEM((CHUNK,), jnp.float32),
        pltpu.VMEM((CHUNK,), jnp.float32),
        pltpu.SemaphoreType.DMA,
        pltpu.SemaphoreType.DMA,
        pltpu.SemaphoreType.DMA,
        pltpu.SemaphoreType.DMA,
        pltpu.VMEM_SHARED((N_PAD, D), jnp.float32),
    ],
)(functools.partial(_sc_scatter_body, sl * SLICE_E))


CBLK = 2000


def _combine_body(*refs):
    out_ref = refs[-1]
    acc = refs[0][...]
    for r in refs[1:-1]:
        acc = acc + r[...]
    out_ref[...] = acc


def _combine(partials):
    specs = []
    args = []
    for p in partials:
        for core in (0, 1):
            specs.append(pl.BlockSpec(
                (None, CBLK, D),
                functools.partial(lambda i, c: (c, i, 0), c=core)))
            args.append(p)
    return pl.pallas_call(
        _combine_body,
        grid=(N // CBLK,),
        in_specs=specs,
        out_specs=pl.BlockSpec((CBLK, D), lambda i: (i, 0)),
        out_shape=jax.ShapeDtypeStruct((N, D), jnp.float32),
    )(*args)


def kernel(msg, x_i, x_j, e_ij, index, num_nodes, W1, b1, W2, b2):
    w1a = W1[:D]
    w1b = W1[D:D + DE]
    w1c = W1[D + DE:]
    b1r = b1.reshape(1, HIDDEN)
    w2r = W2.reshape(1, HIDDEN)
    b2r = b2.reshape(1, 1)
    idx = index.astype(jnp.int32)
    idx4 = jnp.concatenate(
        [idx, jnp.full((E_PAD - E,), N, jnp.int32)]).reshape(
            NSLICES, NC * NS, CHUNKS_PER_W, CHUNK)
    zrows = jnp.zeros((ROWS_PER_TILE, D), jnp.float32)
    partials = []
    for sl in range(NSLICES):
        w = _gate_mlp(sl, x_i, x_j, e_ij, w1a, w1b, w1c, b1r, w2r, b2r)
        partials.append(_sc_scatter_call(sl)(msg, w, idx4[sl], zrows))
    return _combine(partials)


# speedup vs baseline: 1.0486x; 1.0486x over previous
import functools

import jax
import jax.numpy as jnp
from jax import lax
from jax.experimental import pallas as pl
from jax.experimental.pallas import tpu as pltpu
import jax.experimental.pallas.tpu_sc as plsc

E = 320000
N = 10000
D = 128
DE = 16
HIDDEN = 128

NC = 2
NS = 16
CHUNK = 128
CHUNKS_PER_W = 40
EDGES_PER_W = CHUNK * CHUNKS_PER_W
SLICE_E = NC * NS * EDGES_PER_W
NSLICES = 2
E_PAD = NSLICES * SLICE_E
ROWS_PER_TILE = 632
N_PAD = NS * ROWS_PER_TILE
MSG_CLAMP = E - CHUNK
LANE = 16

BLK = 16384


def _mlp_body(xi_ref, xj_ref, e_ref, w1a_ref, w1b_ref, w1c_ref,
              b1_ref, w2_ref, b2_ref, out_ref):
    h = jnp.dot(xj_ref[...], w1a_ref[...], preferred_element_type=jnp.float32)
    h = h + jnp.dot(e_ref[...], w1b_ref[...], preferred_element_type=jnp.float32)
    h = h + jnp.dot(xi_ref[...], w1c_ref[...], preferred_element_type=jnp.float32)
    h = h + b1_ref[...]
    h = jnp.where(h >= 0, h, 0.01 * h)
    rows = [
        lax.dot_general(
            w2_ref[...], h[r * CHUNK:(r + 1) * CHUNK, :],
            (((1,), (1,)), ((), ())), preferred_element_type=jnp.float32)
        for r in range(BLK // CHUNK)
    ]
    w = jnp.concatenate(rows, axis=0) + b2_ref[0, 0]
    out_ref[...] = jnp.tanh(w)


BLKS_PER_SLICE = SLICE_E // BLK


def _gate_mlp(sl, x_i, x_j, e_ij, w1a, w1b, w1c, b1r, w2r, b2r):
    off = sl * BLKS_PER_SLICE
    edge = lambda i: (i + off, 0)
    const = lambda i: (0, 0)
    return pl.pallas_call(
        _mlp_body,
        grid=(BLKS_PER_SLICE,),
        in_specs=[
            pl.BlockSpec((BLK, D), edge),
            pl.BlockSpec((BLK, D), edge),
            pl.BlockSpec((BLK, DE), edge),
            pl.BlockSpec((D, HIDDEN), const),
            pl.BlockSpec((DE, HIDDEN), const),
            pl.BlockSpec((D, HIDDEN), const),
            pl.BlockSpec((1, HIDDEN), const),
            pl.BlockSpec((1, HIDDEN), const),
            pl.BlockSpec((1, 1), const),
        ],
        out_specs=pl.BlockSpec((BLK // CHUNK, CHUNK), lambda i: (i, 0)),
        out_shape=jax.ShapeDtypeStruct((SLICE_E // CHUNK, CHUNK), jnp.float32),
    )(x_i, x_j, e_ij, w1a, w1b, w1c, b1r, w2r, b2r)


def _sc_scatter_body(slice_base, msg_hbm, w_hbm, idx_hbm, zrows_hbm, out_hbm,
                     idxbuf, m0, m1, w0, w1, sm0, sm1, sw0, sw1, accum):
    c = lax.axis_index("c")
    s = lax.axis_index("s")
    wkr = c * NS + s
    pltpu.sync_copy(idx_hbm.at[wkr], idxbuf)
    pltpu.sync_copy(zrows_hbm, accum.at[pl.ds(s * ROWS_PER_TILE, ROWS_PER_TILE)])
    plsc.subcore_barrier()

    row0 = slice_base + wkr * EDGES_PER_W

    def mrow(j):
        return jnp.minimum(row0 + j * CHUNK, MSG_CLAMP)

    def start(j, mbuf, wbuf, msem, wsem):
        pltpu.async_copy(msg_hbm.at[pl.ds(mrow(j), CHUNK)], mbuf, msem)
        pltpu.async_copy(w_hbm.at[wkr * CHUNKS_PER_W + j], wbuf, wsem)

    def wait(j, mbuf, wbuf, msem, wsem):
        pltpu.make_async_copy(
            msg_hbm.at[pl.ds(mrow(j), CHUNK)], mbuf, msem).wait()
        pltpu.make_async_copy(
            w_hbm.at[wkr * CHUNKS_PER_W + j], wbuf, wsem).wait()

    def apply_gate(mbuf, wbuf):
        def group(g, carry):
            r0 = g * LANE
            wv = wbuf[pl.ds(r0, LANE)]
            for i in range(LANE):
                splat = lax.gather(
                    wv, jnp.full((LANE, 1), i, jnp.int32),
                    lax.GatherDimensionNumbers(
                        offset_dims=(), collapsed_slice_dims=(0,),
                        start_index_map=(0,)),
                    (1,), mode=lax.GatherScatterMode.PROMISE_IN_BOUNDS)
                for k in range(D // LANE):
                    sl = pl.ds(k * LANE, LANE)
                    mbuf[r0 + i, sl] = mbuf[r0 + i, sl] * splat
            return carry

        lax.fori_loop(0, CHUNK // LANE, group, 0)

    def scatter(j, mbuf):
        pltpu.sync_copy(mbuf, accum.at[idxbuf.at[j]], add=True)

    start(0, m0, w0, sm0, sw0)

    def body(q, carry):
        j = 2 * q
        start(j + 1, m1, w1, sm1, sw1)
        wait(j, m0, w0, sm0, sw0)
        apply_gate(m0, w0)
        scatter(j, m0)
        start(j + 2, m0, w0, sm0, sw0)
        wait(j + 1, m1, w1, sm1, sw1)
        apply_gate(m1, w1)
        scatter(j + 1, m1)
        return carry

    lax.fori_loop(0, CHUNKS_PER_W // 2 - 1, body, 0)
    start(CHUNKS_PER_W - 1, m1, w1, sm1, sw1)
    wait(CHUNKS_PER_W - 2, m0, w0, sm0, sw0)
    apply_gate(m0, w0)
    scatter(CHUNKS_PER_W - 2, m0)
    wait(CHUNKS_PER_W - 1, m1, w1, sm1, sw1)
    apply_gate(m1, w1)
    scatter(CHUNKS_PER_W - 1, m1)

    plsc.subcore_barrier()
    pltpu.sync_copy(accum.at[pl.ds(s * ROWS_PER_TILE, ROWS_PER_TILE)],
                    out_hbm.at[c, pl.ds(s * ROWS_PER_TILE, ROWS_PER_TILE)])


def _sc_scatter_call(sl):
  return functools.partial(
    pl.kernel,
    out_type=jax.ShapeDtypeStruct((NC, N_PAD, D), jnp.float32),
    mesh=plsc.VectorSubcoreMesh(core_axis_name="c", subcore_axis_name="s"),
    scratch_types=[
        pltpu.VMEM((CHUNKS_PER_W, CHUNK), jnp.int32),
        pltpu.VMEM((CHUNK, D), jnp.float32),
        pltpu.VMEM((CHUNK, D), jnp.float32),
        pltpu.VMEM((CHUNK,), jnp.float32),
        pltpu.VMEM((CHUNK,), jnp.float32),
        pltpu.SemaphoreType.DMA,
        pltpu.SemaphoreType.DMA,
        pltpu.SemaphoreType.DMA,
        pltpu.SemaphoreType.DMA,
        pltpu.VMEM_SHARED((N_PAD, D), jnp.float32),
    ],
)(functools.partial(_sc_scatter_body, sl * SLICE_E))


CBLK = 2000


def _combine_body(*refs):
    out_ref = refs[-1]
    acc = refs[0][...]
    for r in refs[1:-1]:
        acc = acc + r[...]
    out_ref[...] = acc


def _combine(partials):
    specs = []
    args = []
    for p in partials:
        for core in (0, 1):
            specs.append(pl.BlockSpec(
                (None, CBLK, D),
                functools.partial(lambda i, c: (c, i, 0), c=core)))
            args.append(p)
    return pl.pallas_call(
        _combine_body,
        grid=(N // CBLK,),
        in_specs=specs,
        out_specs=pl.BlockSpec((CBLK, D), lambda i: (i, 0)),
        out_shape=jax.ShapeDtypeStruct((N, D), jnp.float32),
    )(*args)


def kernel(msg, x_i, x_j, e_ij, index, num_nodes, W1, b1, W2, b2):
    w1a = W1[:D]
    w1b = W1[D:D + DE]
    w1c = W1[D + DE:]
    b1r = b1.reshape(1, HIDDEN)
    w2r = W2.reshape(1, HIDDEN)
    b2r = b2.reshape(1, 1)
    idx = index.astype(jnp.int32)
    idx4 = jnp.concatenate(
        [idx, jnp.full((E_PAD - E,), N, jnp.int32)]).reshape(
            NSLICES, NC * NS, CHUNKS_PER_W, CHUNK)
    zrows = jnp.zeros((ROWS_PER_TILE, D), jnp.float32)
    partials = []
    for sl in range(NSLICES):
        w = _gate_mlp(sl, x_i, x_j, e_ij, w1a, w1b, w1c, b1r, w2r, b2r)
        partials.append(_sc_scatter_call(sl)(msg, w, idx4[sl], zrows))
    return _combine(partials)


# speedup vs baseline: 1.0649x; 1.0156x over previous
import functools

import jax
import jax.numpy as jnp
from jax import lax
from jax.experimental import pallas as pl
from jax.experimental.pallas import tpu as pltpu
import jax.experimental.pallas.tpu_sc as plsc

E = 320000
N = 10000
D = 128
DE = 16
HIDDEN = 128

NC = 2
NS = 16
CHUNK = 128
SLICE_CHUNKS = (48, 32)
SLICE_E = tuple(NC * NS * CHUNK * cw for cw in SLICE_CHUNKS)
NSLICES = 2
E_PAD = sum(SLICE_E)
ROWS_PER_TILE = 632
N_PAD = NS * ROWS_PER_TILE
MSG_CLAMP = E - CHUNK
LANE = 16

BLK = 16384


def _mlp_body(xi_ref, xj_ref, e_ref, w1a_ref, w1b_ref, w1c_ref,
              b1_ref, w2_ref, b2_ref, out_ref):
    h = jnp.dot(xj_ref[...], w1a_ref[...], preferred_element_type=jnp.float32)
    h = h + jnp.dot(e_ref[...], w1b_ref[...], preferred_element_type=jnp.float32)
    h = h + jnp.dot(xi_ref[...], w1c_ref[...], preferred_element_type=jnp.float32)
    h = h + b1_ref[...]
    h = jnp.where(h >= 0, h, 0.01 * h)
    rows = [
        lax.dot_general(
            w2_ref[...], h[r * CHUNK:(r + 1) * CHUNK, :],
            (((1,), (1,)), ((), ())), preferred_element_type=jnp.float32)
        for r in range(BLK // CHUNK)
    ]
    w = jnp.concatenate(rows, axis=0) + b2_ref[0, 0]
    out_ref[...] = jnp.tanh(w)


def _gate_mlp(sl, x_i, x_j, e_ij, w1a, w1b, w1c, b1r, w2r, b2r):
    off = sum(SLICE_E[:sl]) // BLK
    edge = lambda i: (i + off, 0)
    const = lambda i: (0, 0)
    return pl.pallas_call(
        _mlp_body,
        grid=(SLICE_E[sl] // BLK,),
        in_specs=[
            pl.BlockSpec((BLK, D), edge),
            pl.BlockSpec((BLK, D), edge),
            pl.BlockSpec((BLK, DE), edge),
            pl.BlockSpec((D, HIDDEN), const),
            pl.BlockSpec((DE, HIDDEN), const),
            pl.BlockSpec((D, HIDDEN), const),
            pl.BlockSpec((1, HIDDEN), const),
            pl.BlockSpec((1, HIDDEN), const),
            pl.BlockSpec((1, 1), const),
        ],
        out_specs=pl.BlockSpec((BLK // CHUNK, CHUNK), lambda i: (i, 0)),
        out_shape=jax.ShapeDtypeStruct((SLICE_E[sl] // CHUNK, CHUNK),
                                       jnp.float32),
    )(x_i, x_j, e_ij, w1a, w1b, w1c, b1r, w2r, b2r)


def _sc_scatter_body(slice_base, chunks_per_w, msg_hbm, w_hbm, idx_hbm,
                     zrows_hbm, out_hbm,
                     idxbuf, m0, m1, w0, w1, sm0, sm1, sw0, sw1, accum):
    c = lax.axis_index("c")
    s = lax.axis_index("s")
    wkr = c * NS + s
    pltpu.sync_copy(idx_hbm.at[wkr], idxbuf)
    pltpu.sync_copy(zrows_hbm, accum.at[pl.ds(s * ROWS_PER_TILE, ROWS_PER_TILE)])
    plsc.subcore_barrier()

    row0 = slice_base + wkr * chunks_per_w * CHUNK

    def mrow(j):
        return jnp.minimum(row0 + j * CHUNK, MSG_CLAMP)

    def start(j, mbuf, wbuf, msem, wsem):
        pltpu.async_copy(msg_hbm.at[pl.ds(mrow(j), CHUNK)], mbuf, msem)
        pltpu.async_copy(w_hbm.at[wkr * chunks_per_w + j], wbuf, wsem)

    def wait(j, mbuf, wbuf, msem, wsem):
        pltpu.make_async_copy(
            msg_hbm.at[pl.ds(mrow(j), CHUNK)], mbuf, msem).wait()
        pltpu.make_async_copy(
            w_hbm.at[wkr * chunks_per_w + j], wbuf, wsem).wait()

    def apply_gate(mbuf, wbuf):
        def group(g, carry):
            r0 = g * LANE
            wv = wbuf[pl.ds(r0, LANE)]
            for i in range(LANE):
                splat = lax.gather(
                    wv, jnp.full((LANE, 1), i, jnp.int32),
                    lax.GatherDimensionNumbers(
                        offset_dims=(), collapsed_slice_dims=(0,),
                        start_index_map=(0,)),
                    (1,), mode=lax.GatherScatterMode.PROMISE_IN_BOUNDS)
                for k in range(D // LANE):
                    sl = pl.ds(k * LANE, LANE)
                    mbuf[r0 + i, sl] = mbuf[r0 + i, sl] * splat
            return carry

        lax.fori_loop(0, CHUNK // LANE, group, 0)

    def scatter(j, mbuf):
        pltpu.sync_copy(mbuf, accum.at[idxbuf.at[j]], add=True)

    start(0, m0, w0, sm0, sw0)

    def body(q, carry):
        j = 2 * q
        start(j + 1, m1, w1, sm1, sw1)
        wait(j, m0, w0, sm0, sw0)
        apply_gate(m0, w0)
        scatter(j, m0)
        start(j + 2, m0, w0, sm0, sw0)
        wait(j + 1, m1, w1, sm1, sw1)
        apply_gate(m1, w1)
        scatter(j + 1, m1)
        return carry

    lax.fori_loop(0, chunks_per_w // 2 - 1, body, 0)
    start(chunks_per_w - 1, m1, w1, sm1, sw1)
    wait(chunks_per_w - 2, m0, w0, sm0, sw0)
    apply_gate(m0, w0)
    scatter(chunks_per_w - 2, m0)
    wait(chunks_per_w - 1, m1, w1, sm1, sw1)
    apply_gate(m1, w1)
    scatter(chunks_per_w - 1, m1)

    plsc.subcore_barrier()
    pltpu.sync_copy(accum.at[pl.ds(s * ROWS_PER_TILE, ROWS_PER_TILE)],
                    out_hbm.at[c, pl.ds(s * ROWS_PER_TILE, ROWS_PER_TILE)])


def _sc_scatter_call(sl):
  cw = SLICE_CHUNKS[sl]
  base = sum(SLICE_E[:sl])
  return functools.partial(
    pl.kernel,
    out_type=jax.ShapeDtypeStruct((NC, N_PAD, D), jnp.float32),
    mesh=plsc.VectorSubcoreMesh(core_axis_name="c", subcore_axis_name="s"),
    scratch_types=[
        pltpu.VMEM((cw, CHUNK), jnp.int32),
        pltpu.VMEM((CHUNK, D), jnp.float32),
        pltpu.VMEM((CHUNK, D), jnp.float32),
        pltpu.VMEM((CHUNK,), jnp.float32),
        pltpu.VMEM((CHUNK,), jnp.float32),
        pltpu.SemaphoreType.DMA,
        pltpu.SemaphoreType.DMA,
        pltpu.SemaphoreType.DMA,
        pltpu.SemaphoreType.DMA,
        pltpu.VMEM_SHARED((N_PAD, D), jnp.float32),
    ],
)(functools.partial(_sc_scatter_body, base, cw))


CBLK = 2000


def _combine_body(*refs):
    out_ref = refs[-1]
    acc = refs[0][...]
    for r in refs[1:-1]:
        acc = acc + r[...]
    out_ref[...] = acc


def _combine(partials):
    specs = []
    args = []
    for p in partials:
        for core in (0, 1):
            specs.append(pl.BlockSpec(
                (None, CBLK, D),
                functools.partial(lambda i, c: (c, i, 0), c=core)))
            args.append(p)
    return pl.pallas_call(
        _combine_body,
        grid=(N // CBLK,),
        in_specs=specs,
        out_specs=pl.BlockSpec((CBLK, D), lambda i: (i, 0)),
        out_shape=jax.ShapeDtypeStruct((N, D), jnp.float32),
    )(*args)


def kernel(msg, x_i, x_j, e_ij, index, num_nodes, W1, b1, W2, b2):
    w1a = W1[:D]
    w1b = W1[D:D + DE]
    w1c = W1[D + DE:]
    b1r = b1.reshape(1, HIDDEN)
    w2r = W2.reshape(1, HIDDEN)
    b2r = b2.reshape(1, 1)
    idx = index.astype(jnp.int32)
    idx_pad = jnp.concatenate([idx, jnp.full((E_PAD - E,), N, jnp.int32)])
    zrows = jnp.zeros((ROWS_PER_TILE, D), jnp.float32)
    partials = []
    base = 0
    for sl in range(NSLICES):
        idx_s = idx_pad[base:base + SLICE_E[sl]].reshape(
            NC * NS, SLICE_CHUNKS[sl], CHUNK)
        base += SLICE_E[sl]
        w = _gate_mlp(sl, x_i, x_j, e_ij, w1a, w1b, w1c, b1r, w2r, b2r)
        partials.append(_sc_scatter_call(sl)(msg, w, idx_s, zrows))
    return _combine(partials)


# speedup vs baseline: 1.0824x; 1.0164x over previous
import functools

import jax
import jax.numpy as jnp
from jax import lax
from jax.experimental import pallas as pl
from jax.experimental.pallas import tpu as pltpu
import jax.experimental.pallas.tpu_sc as plsc

E = 320000
N = 10000
D = 128
DE = 16
HIDDEN = 128

NC = 2
NS = 16
CHUNK = 128
SLICE_CHUNKS = (48, 32)
SLICE_E = tuple(NC * NS * CHUNK * cw for cw in SLICE_CHUNKS)
NSLICES = 2
E_PAD = sum(SLICE_E)
ROWS_PER_TILE = 632
N_PAD = NS * ROWS_PER_TILE
MSG_CLAMP = E - CHUNK
LANE = 16

BLK = 16384


def _mlp_body(xi_ref, xj_ref, e_ref, w1a_ref, w1b_ref, w1c_ref,
              b1_ref, w2_ref, b2_ref, out_ref):
    h = jnp.dot(xj_ref[...], w1a_ref[...], preferred_element_type=jnp.float32)
    h = h + jnp.dot(e_ref[...], w1b_ref[...], preferred_element_type=jnp.float32)
    h = h + jnp.dot(xi_ref[...], w1c_ref[...], preferred_element_type=jnp.float32)
    h = h + b1_ref[...]
    h = jnp.where(h >= 0, h, 0.01 * h)
    rows = [
        lax.dot_general(
            w2_ref[...], h[r * CHUNK:(r + 1) * CHUNK, :],
            (((1,), (1,)), ((), ())), preferred_element_type=jnp.float32)
        for r in range(BLK // CHUNK)
    ]
    w = jnp.concatenate(rows, axis=0) + b2_ref[0, 0]
    out_ref[...] = jnp.tanh(w)


def _gate_mlp(sl, x_i, x_j, e_ij, w1a, w1b, w1c, b1r, w2r, b2r):
    off = sum(SLICE_E[:sl]) // BLK
    edge = lambda i: (i + off, 0)
    const = lambda i: (0, 0)
    return pl.pallas_call(
        _mlp_body,
        grid=(SLICE_E[sl] // BLK,),
        in_specs=[
            pl.BlockSpec((BLK, D), edge),
            pl.BlockSpec((BLK, D), edge),
            pl.BlockSpec((BLK, DE), edge),
            pl.BlockSpec((D, HIDDEN), const),
            pl.BlockSpec((DE, HIDDEN), const),
            pl.BlockSpec((D, HIDDEN), const),
            pl.BlockSpec((1, HIDDEN), const),
            pl.BlockSpec((1, HIDDEN), const),
            pl.BlockSpec((1, 1), const),
        ],
        out_specs=pl.BlockSpec((BLK // CHUNK, CHUNK), lambda i: (i, 0)),
        out_shape=jax.ShapeDtypeStruct((SLICE_E[sl] // CHUNK, CHUNK),
                                       jnp.float32),
    )(x_i, x_j, e_ij, w1a, w1b, w1c, b1r, w2r, b2r)


def _sc_scatter_body(slice_base, chunks_per_w, init_per_core, msg_hbm, w_hbm,
                     idx_hbm, init_hbm, out_hbm,
                     idxbuf, m0, m1, w0, w1, sm0, sm1, sw0, sw1, accum):
    c = lax.axis_index("c")
    s = lax.axis_index("s")
    wkr = c * NS + s
    stripe = pl.ds(s * ROWS_PER_TILE, ROWS_PER_TILE)
    pltpu.async_copy(idx_hbm.at[wkr], idxbuf, sm0)
    if init_per_core:
        pltpu.async_copy(init_hbm.at[c, stripe], accum.at[stripe], sm1)
        pltpu.make_async_copy(init_hbm.at[c, stripe], accum.at[stripe],
                              sm1).wait()
    else:
        pltpu.async_copy(init_hbm.at[stripe], accum.at[stripe], sm1)
        pltpu.make_async_copy(init_hbm.at[stripe], accum.at[stripe],
                              sm1).wait()
    pltpu.make_async_copy(idx_hbm.at[wkr], idxbuf, sm0).wait()
    plsc.subcore_barrier()

    row0 = slice_base + wkr * chunks_per_w * CHUNK

    def mrow(j):
        return jnp.minimum(row0 + j * CHUNK, MSG_CLAMP)

    def start(j, mbuf, wbuf, msem, wsem):
        pltpu.async_copy(msg_hbm.at[pl.ds(mrow(j), CHUNK)], mbuf, msem)
        pltpu.async_copy(w_hbm.at[wkr * chunks_per_w + j], wbuf, wsem)

    def wait(j, mbuf, wbuf, msem, wsem):
        pltpu.make_async_copy(
            msg_hbm.at[pl.ds(mrow(j), CHUNK)], mbuf, msem).wait()
        pltpu.make_async_copy(
            w_hbm.at[wkr * chunks_per_w + j], wbuf, wsem).wait()

    def apply_gate(mbuf, wbuf):
        def group(g, carry):
            r0 = g * LANE
            wv = wbuf[pl.ds(r0, LANE)]
            for i in range(LANE):
                splat = lax.gather(
                    wv, jnp.full((LANE, 1), i, jnp.int32),
                    lax.GatherDimensionNumbers(
                        offset_dims=(), collapsed_slice_dims=(0,),
                        start_index_map=(0,)),
                    (1,), mode=lax.GatherScatterMode.PROMISE_IN_BOUNDS)
                for k in range(D // LANE):
                    sl = pl.ds(k * LANE, LANE)
                    mbuf[r0 + i, sl] = mbuf[r0 + i, sl] * splat
            return carry

        lax.fori_loop(0, CHUNK // LANE, group, 0)

    def scatter(j, mbuf):
        pltpu.sync_copy(mbuf, accum.at[idxbuf.at[j]], add=True)

    start(0, m0, w0, sm0, sw0)

    def body(q, carry):
        j = 2 * q
        start(j + 1, m1, w1, sm1, sw1)
        wait(j, m0, w0, sm0, sw0)
        apply_gate(m0, w0)
        scatter(j, m0)
        start(j + 2, m0, w0, sm0, sw0)
        wait(j + 1, m1, w1, sm1, sw1)
        apply_gate(m1, w1)
        scatter(j + 1, m1)
        return carry

    lax.fori_loop(0, chunks_per_w // 2 - 1, body, 0)
    start(chunks_per_w - 1, m1, w1, sm1, sw1)
    wait(chunks_per_w - 2, m0, w0, sm0, sw0)
    apply_gate(m0, w0)
    scatter(chunks_per_w - 2, m0)
    wait(chunks_per_w - 1, m1, w1, sm1, sw1)
    apply_gate(m1, w1)
    scatter(chunks_per_w - 1, m1)

    plsc.subcore_barrier()
    pltpu.sync_copy(accum.at[pl.ds(s * ROWS_PER_TILE, ROWS_PER_TILE)],
                    out_hbm.at[c, pl.ds(s * ROWS_PER_TILE, ROWS_PER_TILE)])


def _sc_scatter_call(sl):
  cw = SLICE_CHUNKS[sl]
  base = sum(SLICE_E[:sl])
  return functools.partial(
    pl.kernel,
    out_type=jax.ShapeDtypeStruct((NC, N_PAD, D), jnp.float32),
    mesh=plsc.VectorSubcoreMesh(core_axis_name="c", subcore_axis_name="s"),
    scratch_types=[
        pltpu.VMEM((cw, CHUNK), jnp.int32),
        pltpu.VMEM((CHUNK, D), jnp.float32),
        pltpu.VMEM((CHUNK, D), jnp.float32),
        pltpu.VMEM((CHUNK,), jnp.float32),
        pltpu.VMEM((CHUNK,), jnp.float32),
        pltpu.SemaphoreType.DMA,
        pltpu.SemaphoreType.DMA,
        pltpu.SemaphoreType.DMA,
        pltpu.SemaphoreType.DMA,
        pltpu.VMEM_SHARED((N_PAD, D), jnp.float32),
    ],
)(functools.partial(_sc_scatter_body, base, cw, sl > 0))


CBLK = 2000


def _combine_body(*refs):
    out_ref = refs[-1]
    acc = refs[0][...]
    for r in refs[1:-1]:
        acc = acc + r[...]
    out_ref[...] = acc


def _combine(partials):
    specs = []
    args = []
    for p in partials:
        for core in (0, 1):
            specs.append(pl.BlockSpec(
                (None, CBLK, D),
                functools.partial(lambda i, c: (c, i, 0), c=core)))
            args.append(p)
    return pl.pallas_call(
        _combine_body,
        grid=(N // CBLK,),
        in_specs=specs,
        out_specs=pl.BlockSpec((CBLK, D), lambda i: (i, 0)),
        out_shape=jax.ShapeDtypeStruct((N, D), jnp.float32),
    )(*args)


def kernel(msg, x_i, x_j, e_ij, index, num_nodes, W1, b1, W2, b2):
    w1a = W1[:D]
    w1b = W1[D:D + DE]
    w1c = W1[D + DE:]
    b1r = b1.reshape(1, HIDDEN)
    w2r = W2.reshape(1, HIDDEN)
    b2r = b2.reshape(1, 1)
    idx = index.astype(jnp.int32)
    idx_pad = jnp.concatenate([idx, jnp.full((E_PAD - E,), N, jnp.int32)])
    init = jnp.zeros((N_PAD, D), jnp.float32)
    base = 0
    for sl in range(NSLICES):
        idx_s = idx_pad[base:base + SLICE_E[sl]].reshape(
            NC * NS, SLICE_CHUNKS[sl], CHUNK)
        base += SLICE_E[sl]
        w = _gate_mlp(sl, x_i, x_j, e_ij, w1a, w1b, w1c, b1r, w2r, b2r)
        init = _sc_scatter_call(sl)(msg, w, idx_s, init)
    return _combine([init])
